# baseline (speedup 1.0000x reference)
import functools

import jax
import jax.numpy as jnp
from jax import lax
from jax.experimental import pallas as pl
from jax.experimental.pallas import tpu as pltpu
from jax.experimental.pallas import tpu_sc as plsc

B, L = 4096, 200
D = 64
LANES = 16
NGRP = D // LANES

NC, NS = 2, 16
NW = NC * NS
BPW = B // NW
CHUNK = L // 2
NCHUNK = 2 * BPW
NBUF = 4


def _sc_embed_sum(x2, table):
    mesh = plsc.VectorSubcoreMesh(core_axis_name="c", subcore_axis_name="s")

    @functools.partial(
        pl.kernel,
        out_type=jax.ShapeDtypeStruct((B, D), jnp.float32),
        mesh=mesh,
        compiler_params=pltpu.CompilerParams(use_tc_tiling_on_sc=False),
        scratch_types=[
            pltpu.VMEM((NCHUNK, CHUNK), jnp.int32),
            pltpu.VMEM((NBUF, CHUNK, D), jnp.float32),
            pltpu.VMEM((BPW, D), jnp.float32),
        ] + [pltpu.SemaphoreType.DMA] * NBUF,
    )
    def k(x_hbm, table_hbm, out_hbm, idx_v, rows_v, out_v, *sems):
        wid = lax.axis_index("s") * NC + lax.axis_index("c")
        base = wid * NCHUNK
        pltpu.sync_copy(x_hbm.at[pl.ds(base, NCHUNK)], idx_v)

        def start(j, p):
            return pltpu.async_copy(
                table_hbm.at[idx_v.at[j]], rows_v.at[p], sems[p])

        for p in range(NBUF):
            start(p, p)

        def sum_chunk(p, accs):
            buf = rows_v.at[p]

            def rbody(r, a):
                return tuple(
                    a[g] + buf[r, pl.ds(g * LANES, LANES)]
                    for g in range(NGRP))

            return lax.fori_loop(0, CHUNK, rbody, accs, unroll=2)

        def outer(jj):
            for b in range(NBUF // 2):
                accs = tuple(jnp.zeros((LANES,), jnp.float32)
                             for _ in range(NGRP))
                for h in range(2):
                    p = 2 * b + h
                    j = jj + p
                    pltpu.make_async_copy(
                        table_hbm.at[idx_v.at[j]], rows_v.at[p], sems[p]
                    ).wait()
                    accs = sum_chunk(p, accs)

                    @pl.when(j + NBUF < NCHUNK)
                    def _():
                        start(j + NBUF, p)

                row = jj // 2 + b
                for g in range(NGRP):
                    out_v[row, pl.ds(g * LANES, LANES)] = accs[g]

        pl.loop(0, NCHUNK, step=NBUF)(outer)
        pltpu.sync_copy(out_v, out_hbm.at[pl.ds(wid * BPW, BPW)])

    return k(x2, table)


def _tc_mlp(h_sum, W1t, b1, W2t, b2, W3t, b3):
    def body(h_ref, w1_ref, b1_ref, w2_ref, b2_ref, w3_ref, b3_ref, o_ref):
        h = h_ref[...] * (1.0 / L)
        h = jnp.dot(h, w1_ref[...], preferred_element_type=jnp.float32)
        h = jnp.maximum(h + b1_ref[...], 0.0)
        h = jnp.dot(h, w2_ref[...], preferred_element_type=jnp.float32)
        h = jnp.maximum(h + b2_ref[...], 0.0)
        o = jnp.dot(h, w3_ref[...], preferred_element_type=jnp.float32)
        o_ref[...] = o + b3_ref[...]

    return pl.pallas_call(
        body,
        out_shape=jax.ShapeDtypeStruct((B, W3t.shape[1]), jnp.float32),
    )(h_sum, W1t, b1[None, :], W2t, b2[None, :], W3t, b3[None, :])


def kernel(x, table, W1, b1, W2, b2, W3, b3):
    x2 = x.reshape(2 * B, L // 2)
    h_sum = _sc_embed_sum(x2, table)
    return _tc_mlp(h_sum, W1.T, b1, W2.T, b2, W3.T, b3)
